# baseline (speedup 1.0000x reference)
import functools

import jax
import jax.numpy as jnp
from jax import lax
from jax.experimental import pallas as pl
from jax.experimental.pallas import tpu as pltpu
from jax.experimental.pallas import tpu_sc as plsc

_INFO = plsc.get_sparse_core_info()
_NC, _NS, _L = _INFO.num_cores, _INFO.num_subcores, _INFO.num_lanes
_NW = _NC * _NS

_BATCH = 16384
_CHUNK = _BATCH // _NW
_N_VENDORS = 1000000
_VTAB_PAD = 1000448


def _fe_kernel(vidx_hbm, widx_hbm, lc_hbm, vtab_hbm, wtab_hbm, beta_hbm,
               out_hbm, vidx_v, widx_v, lc_v, veff_v, weff_v, beta_v, out_v,
               sem):
    wid = lax.axis_index("s") * _NC + lax.axis_index("c")

    pltpu.sync_copy(vidx_hbm.at[wid], vidx_v)
    pltpu.sync_copy(widx_hbm.at[wid], widx_v)
    gathers = [
        pltpu.async_copy(vtab_hbm.at[vidx_v], veff_v, sem),
        pltpu.async_copy(wtab_hbm.at[widx_v], weff_v, sem),
    ]

    pltpu.sync_copy(lc_hbm.at[wid], lc_v)
    pltpu.sync_copy(beta_hbm, beta_v)
    for g in gathers:
        g.wait()

    b16 = beta_v[...]
    for c in range(_CHUNK // _L):
        sl = pl.ds(c * _L, _L)
        out_v[sl] = lc_v[sl] * b16 + veff_v[sl] + weff_v[sl]

    pltpu.sync_copy(out_v, out_hbm.at[wid])


@jax.jit
def _run(vidx2, widx2, lc2, vtab, wtab, beta16):
    mesh = plsc.VectorSubcoreMesh(core_axis_name="c", subcore_axis_name="s")
    f = functools.partial(
        pl.kernel,
        out_type=jax.ShapeDtypeStruct((_NW, _CHUNK), jnp.float32),
        mesh=mesh,
        scratch_types=[
            pltpu.VMEM((_CHUNK,), jnp.int32),
            pltpu.VMEM((_CHUNK,), jnp.int32),
            pltpu.VMEM((_CHUNK,), jnp.float32),
            pltpu.VMEM((_CHUNK,), jnp.float32),
            pltpu.VMEM((_CHUNK,), jnp.float32),
            pltpu.VMEM((_L,), jnp.float32),
            pltpu.VMEM((_CHUNK,), jnp.float32),
            pltpu.SemaphoreType.DMA,
        ],
    )(_fe_kernel)
    return f(vidx2, widx2, lc2, vtab, wtab, beta16)


def kernel(vendor_ids, week_ids, log_clicks, vendor_fe, week_fe, beta):
    vidx2 = vendor_ids.astype(jnp.int32).reshape(_NW, _CHUNK)
    widx2 = week_ids.astype(jnp.int32).reshape(_NW, _CHUNK)
    lc2 = log_clicks.reshape(_NW, _CHUNK)
    vtab = jnp.pad(vendor_fe, ((0, _VTAB_PAD - _N_VENDORS), (0, 0))).reshape(-1)
    wtab = week_fe.reshape(-1)
    beta16 = jnp.broadcast_to(beta.astype(jnp.float32), (_L,))
    out2 = _run(vidx2, widx2, lc2, vtab, wtab, beta16)
    return out2.reshape(_BATCH)
